# baseline (speedup 1.0000x reference)
import functools

import jax
import jax.numpy as jnp
from jax import lax
from jax.experimental import pallas as pl
from jax.experimental.pallas import tpu as pltpu

TOPK = 50
TOPP = 0.9
NEG = float("-inf")
GMAX = 64
RB = 8
NSUB = 8
RFAST = 3
RCAP = 56
RCHK = 8


def _group_extract(vals, cnts, nr):
    giota = lax.broadcasted_iota(jnp.int32, (nr, 1, GMAX), 2)

    def body2(j, st):
        v, gv, gc, tot = st
        active = tot < TOPK
        m = jnp.max(v, axis=(1, 2), keepdims=True)
        c = jnp.sum(jnp.where(v == m, cnts, 0.0), axis=(1, 2), keepdims=True)
        rec = jnp.logical_and(giota == j, active)
        gv = jnp.where(rec, m, gv)
        gc = jnp.where(rec, c, gc)
        v = jnp.where(jnp.logical_and(v == m, active), NEG, v)
        return v, gv, gc, tot + jnp.where(active, c, 0.0)

    gv0 = jnp.full((nr, 1, GMAX), NEG, dtype=jnp.float32)
    gc0 = jnp.zeros((nr, 1, GMAX), dtype=jnp.float32)
    tot0 = jnp.zeros((nr, 1, 1), dtype=jnp.float32)
    _, gv, gc, _ = lax.fori_loop(
        0, TOPK, body2, (vals, gv0, gc0, tot0), unroll=4
    )
    return gv, gc


def _fused_kernel(x_ref, o_ref, a_ref, vals_ref, cnts_ref, *, nr, v, nrows, ncols):
    x = x_ref[...]
    nb = nr * NSUB
    sub = nrows // NSUB
    vmain = (v // ncols) * ncols
    ntail = v - vmain
    parts = [x[:, :vmain].reshape(nr, vmain // ncols, ncols)]
    nfill = nrows - vmain // ncols
    if ntail:
        parts.append(
            jnp.concatenate(
                [x[:, vmain:], jnp.full((nr, ncols - ntail), NEG, jnp.float32)],
                axis=1,
            ).reshape(nr, 1, ncols)
        )
        nfill -= 1
    if nfill:
        parts.append(jnp.full((nr, nfill, ncols), NEG, jnp.float32))
    xa = jnp.concatenate(parts, axis=1)
    ab = xa.reshape(nb, sub, ncols)
    a_ref[...] = ab
    zero_sub = jnp.zeros((nr, NSUB), jnp.float32)
    vals_ref[...] = jnp.full((nb, RCAP, ncols), NEG, jnp.float32)
    cnts_ref[...] = jnp.zeros((nb, RCAP, ncols), jnp.float32)

    def cond1(st):
        _, r, done = st
        return jnp.logical_and(r < TOPK, jnp.sum(done) < nr)

    def body1(st):
        cm, r, done = st
        a = a_ref[...]
        eq = a == cm
        cnt = jnp.sum(eq.astype(jnp.float32), axis=1, keepdims=True)
        vals_ref[:, pl.ds(jnp.minimum(r, RCAP - 1), 1), :] = cm
        cnts_ref[:, pl.ds(jnp.minimum(r, RCAP - 1), 1), :] = cnt
        a = jnp.where(eq, NEG, a)
        a_ref[...] = a
        cm = jnp.max(a, axis=1, keepdims=True)
        m_row = jnp.max(
            jnp.max(cm, axis=2).reshape(nr, NSUB), axis=1, keepdims=True
        ).reshape(nr, 1, 1)
        m_b = (m_row.reshape(nr, 1) + zero_sub).reshape(nb, 1, 1)
        above = jnp.sum(
            jnp.where(vals_ref[:, :RCHK, :] > m_b, cnts_ref[:, :RCHK, :], 0.0),
            axis=(1, 2),
            keepdims=True,
        )
        above_row = jnp.sum(above.reshape(nr, NSUB), axis=1, keepdims=True)
        done = (above_row >= TOPK).astype(jnp.float32)
        return cm, r + 1, done

    cm0 = jnp.max(ab, axis=1, keepdims=True)
    done0 = jnp.zeros((nr, 1), dtype=jnp.float32)
    _, rstop, _ = lax.while_loop(
        cond1, body1, (cm0, jnp.int32(0), done0)
    )

    gv, gc = lax.cond(
        rstop <= RFAST,
        lambda: _group_extract(
            vals_ref[:, :RFAST, :].reshape(nr, NSUB * RFAST, 128),
            cnts_ref[:, :RFAST, :].reshape(nr, NSUB * RFAST, 128),
            nr,
        ),
        lambda: _group_extract(
            vals_ref[...].reshape(nr, NSUB * RCAP, 128),
            cnts_ref[...].reshape(nr, NSUB * RCAP, 128),
            nr,
        ),
    )

    gvalid = gc > 0.0
    m_top = jnp.max(gv, axis=2, keepdims=True)
    w = jnp.where(gvalid, jnp.exp(gv - m_top), 0.0)
    mass = gc * w
    s_total = jnp.sum(mass, axis=2, keepdims=True)
    tri = (
        lax.broadcasted_iota(jnp.int32, (GMAX, GMAX), 0)
        <= lax.broadcasted_iota(jnp.int32, (GMAX, GMAX), 1)
    ).astype(jnp.float32)
    cum = jnp.dot(
        mass.reshape(nr, GMAX), tri, preferred_element_type=jnp.float32
    ).reshape(nr, 1, GMAX)
    prev = cum - mass
    thr = TOPP * s_total
    nk = jnp.floor((thr - prev) / w) + 1.0
    nk = jnp.where(w > 0.0, nk, jnp.where(prev <= thr, gc, 0.0))
    nk = jnp.where(gvalid, jnp.clip(nk, 0.0, gc), 0.0)
    kept = nk >= 1.0
    t_p = jnp.min(jnp.where(kept, gv, jnp.inf), axis=2, keepdims=True)
    n_at = jnp.sum(
        jnp.where(jnp.logical_and(kept, gv == t_p), nk, 0.0),
        axis=2,
        keepdims=True,
    )
    c_at = jnp.sum(jnp.where(gv == t_p, gc, 0.0), axis=2, keepdims=True)
    lse = m_top + jnp.log(jnp.sum(nk * w, axis=2, keepdims=True))

    split = n_at < c_at

    def icut_split():
        xb = xa
        eq = xb == t_p
        eqf = eq.astype(jnp.float32)
        tri_c = (
            lax.broadcasted_iota(jnp.int32, (ncols, ncols), 0)
            <= lax.broadcasted_iota(jnp.int32, (ncols, ncols), 1)
        ).astype(jnp.float32)
        incol = jnp.stack(
            [
                jnp.dot(eqf[i], tri_c, preferred_element_type=jnp.float32)
                for i in range(nr)
            ],
            axis=0,
        )
        rowtot = jnp.sum(eqf, axis=2)
        tri_r = (
            lax.broadcasted_iota(jnp.int32, (nrows, nrows), 0)
            < lax.broadcasted_iota(jnp.int32, (nrows, nrows), 1)
        ).astype(jnp.float32)
        rowprev = jnp.dot(
            rowtot, tri_r, preferred_element_type=jnp.float32
        ).reshape(nr, nrows, 1)
        pc = incol + rowprev
        hit = jnp.logical_and(eq, pc == n_at)
        flat = lax.broadcasted_iota(
            jnp.int32, (nr, nrows, ncols), 1
        ) * ncols + lax.broadcasted_iota(jnp.int32, (nr, nrows, ncols), 2)
        icr = jnp.max(jnp.where(hit, flat, -1), axis=(1, 2), keepdims=True)
        return jnp.where(split, icr, 2**30)

    icut = lax.cond(
        jnp.any(split),
        icut_split,
        lambda: jnp.full((nr, 1, 1), 2**30, jnp.int32),
    )

    tp2 = t_p.reshape(nr, 1)
    lse2 = lse.reshape(nr, 1)
    ic2 = icut.reshape(nr, 1)
    vi = lax.broadcasted_iota(jnp.int32, (nr, v), 1)
    keep = jnp.logical_or(x > tp2, jnp.logical_and(x == tp2, vi <= ic2))
    o_ref[...] = jnp.where(keep, x - lse2, NEG)


@jax.jit
def kernel(logits):
    b, h, v = logits.shape
    n = b * h
    nrows = (v + 127) // 128
    nrows = ((nrows + 8 * NSUB - 1) // (8 * NSUB)) * (8 * NSUB)
    x2 = logits.reshape(n, v)
    rb = RB if n % RB == 0 else 1
    out = pl.pallas_call(
        functools.partial(_fused_kernel, nr=rb, v=v, nrows=nrows, ncols=128),
        grid=(n // rb,),
        in_specs=[pl.BlockSpec((rb, v), lambda i: (i, 0))],
        out_specs=pl.BlockSpec((rb, v), lambda i: (i, 0)),
        out_shape=jax.ShapeDtypeStruct((n, v), jnp.float32),
        scratch_shapes=[
            pltpu.VMEM((rb * NSUB, nrows // NSUB, 128), jnp.float32),
            pltpu.VMEM((rb * NSUB, RCAP, 128), jnp.float32),
            pltpu.VMEM((rb * NSUB, RCAP, 128), jnp.float32),
        ],
    )(x2)
    return out.reshape(b, h, v)


# speedup vs baseline: 1.4088x; 1.4088x over previous
import functools

import jax
import jax.numpy as jnp
from jax import lax
from jax.experimental import pallas as pl
from jax.experimental.pallas import tpu as pltpu

TOPK = 50
TOPP = 0.9
NEG = float("-inf")
GMAX = 64
RB = 8
NSUB = 4
RFAST = 3
RCAP = 56
RCHK = 8


def _group_extract(vals, cnts, nr):
    giota = lax.broadcasted_iota(jnp.int32, (nr, 1, GMAX), 2)

    def body2(j, st):
        v, gv, gc, tot = st
        active = tot < TOPK
        m = jnp.max(v, axis=(1, 2), keepdims=True)
        c = jnp.sum(jnp.where(v == m, cnts, 0.0), axis=(1, 2), keepdims=True)
        rec = jnp.logical_and(giota == j, active)
        gv = jnp.where(rec, m, gv)
        gc = jnp.where(rec, c, gc)
        v = jnp.where(jnp.logical_and(v == m, active), NEG, v)
        return v, gv, gc, tot + jnp.where(active, c, 0.0)

    gv0 = jnp.full((nr, 1, GMAX), NEG, dtype=jnp.float32)
    gc0 = jnp.zeros((nr, 1, GMAX), dtype=jnp.float32)
    tot0 = jnp.zeros((nr, 1, 1), dtype=jnp.float32)
    _, gv, gc, _ = lax.fori_loop(
        0, TOPK, body2, (vals, gv0, gc0, tot0), unroll=4
    )
    return gv, gc


def _fused_kernel(x_ref, o_ref, a_ref, vals_ref, cnts_ref, *, nr, v, nrows, ncols):
    x = x_ref[...]
    nb = nr * NSUB
    sub = nrows // NSUB
    vmain = (v // ncols) * ncols
    ntail = v - vmain
    parts = [x[:, :vmain].reshape(nr, vmain // ncols, ncols)]
    nfill = nrows - vmain // ncols
    if ntail:
        parts.append(
            jnp.concatenate(
                [x[:, vmain:], jnp.full((nr, ncols - ntail), NEG, jnp.float32)],
                axis=1,
            ).reshape(nr, 1, ncols)
        )
        nfill -= 1
    if nfill:
        parts.append(jnp.full((nr, nfill, ncols), NEG, jnp.float32))
    xa = jnp.concatenate(parts, axis=1)
    ab = xa.reshape(nb, sub, ncols)
    a_ref[...] = ab
    zero_sub = jnp.zeros((nr, NSUB), jnp.float32)
    vals_ref[...] = jnp.full((nb, RCAP, ncols), NEG, jnp.float32)
    cnts_ref[...] = jnp.zeros((nb, RCAP, ncols), jnp.float32)

    def cond1(st):
        _, r, done = st
        return jnp.logical_and(r < TOPK, jnp.sum(done) < nr)

    def body1(st):
        cm, r, done = st
        a = a_ref[...]
        eq = a == cm
        cnt = jnp.sum(eq.astype(jnp.float32), axis=1, keepdims=True)
        vals_ref[:, pl.ds(jnp.minimum(r, RCAP - 1), 1), :] = cm
        cnts_ref[:, pl.ds(jnp.minimum(r, RCAP - 1), 1), :] = cnt
        a = jnp.where(eq, NEG, a)
        a_ref[...] = a
        cm = jnp.max(a, axis=1, keepdims=True)
        m_row = jnp.max(
            jnp.max(cm, axis=2).reshape(nr, NSUB), axis=1, keepdims=True
        ).reshape(nr, 1, 1)
        m_b = (m_row.reshape(nr, 1) + zero_sub).reshape(nb, 1, 1)
        above = jnp.sum(
            jnp.where(vals_ref[:, :RCHK, :] > m_b, cnts_ref[:, :RCHK, :], 0.0),
            axis=(1, 2),
            keepdims=True,
        )
        above_row = jnp.sum(above.reshape(nr, NSUB), axis=1, keepdims=True)
        done = (above_row >= TOPK).astype(jnp.float32)
        return cm, r + 1, done

    cm0 = jnp.max(ab, axis=1, keepdims=True)
    done0 = jnp.zeros((nr, 1), dtype=jnp.float32)
    _, rstop, _ = lax.while_loop(
        cond1, body1, (cm0, jnp.int32(0), done0)
    )

    gv, gc = lax.cond(
        rstop <= RFAST,
        lambda: _group_extract(
            vals_ref[:, :RFAST, :].reshape(nr, NSUB * RFAST, 128),
            cnts_ref[:, :RFAST, :].reshape(nr, NSUB * RFAST, 128),
            nr,
        ),
        lambda: _group_extract(
            vals_ref[...].reshape(nr, NSUB * RCAP, 128),
            cnts_ref[...].reshape(nr, NSUB * RCAP, 128),
            nr,
        ),
    )

    gvalid = gc > 0.0
    m_top = jnp.max(gv, axis=2, keepdims=True)
    w = jnp.where(gvalid, jnp.exp(gv - m_top), 0.0)
    mass = gc * w
    s_total = jnp.sum(mass, axis=2, keepdims=True)
    tri = (
        lax.broadcasted_iota(jnp.int32, (GMAX, GMAX), 0)
        <= lax.broadcasted_iota(jnp.int32, (GMAX, GMAX), 1)
    ).astype(jnp.float32)
    cum = jnp.dot(
        mass.reshape(nr, GMAX), tri, preferred_element_type=jnp.float32
    ).reshape(nr, 1, GMAX)
    prev = cum - mass
    thr = TOPP * s_total
    nk = jnp.floor((thr - prev) / w) + 1.0
    nk = jnp.where(w > 0.0, nk, jnp.where(prev <= thr, gc, 0.0))
    nk = jnp.where(gvalid, jnp.clip(nk, 0.0, gc), 0.0)
    kept = nk >= 1.0
    t_p = jnp.min(jnp.where(kept, gv, jnp.inf), axis=2, keepdims=True)
    n_at = jnp.sum(
        jnp.where(jnp.logical_and(kept, gv == t_p), nk, 0.0),
        axis=2,
        keepdims=True,
    )
    c_at = jnp.sum(jnp.where(gv == t_p, gc, 0.0), axis=2, keepdims=True)
    lse = m_top + jnp.log(jnp.sum(nk * w, axis=2, keepdims=True))

    split = n_at < c_at

    def icut_split():
        xb = xa
        eq = xb == t_p
        eqf = eq.astype(jnp.float32)
        tri_c = (
            lax.broadcasted_iota(jnp.int32, (ncols, ncols), 0)
            <= lax.broadcasted_iota(jnp.int32, (ncols, ncols), 1)
        ).astype(jnp.float32)
        incol = jnp.stack(
            [
                jnp.dot(eqf[i], tri_c, preferred_element_type=jnp.float32)
                for i in range(nr)
            ],
            axis=0,
        )
        rowtot = jnp.sum(eqf, axis=2)
        tri_r = (
            lax.broadcasted_iota(jnp.int32, (nrows, nrows), 0)
            < lax.broadcasted_iota(jnp.int32, (nrows, nrows), 1)
        ).astype(jnp.float32)
        rowprev = jnp.dot(
            rowtot, tri_r, preferred_element_type=jnp.float32
        ).reshape(nr, nrows, 1)
        pc = incol + rowprev
        hit = jnp.logical_and(eq, pc == n_at)
        flat = lax.broadcasted_iota(
            jnp.int32, (nr, nrows, ncols), 1
        ) * ncols + lax.broadcasted_iota(jnp.int32, (nr, nrows, ncols), 2)
        icr = jnp.max(jnp.where(hit, flat, -1), axis=(1, 2), keepdims=True)
        return jnp.where(split, icr, 2**30)

    icut = lax.cond(
        jnp.any(split),
        icut_split,
        lambda: jnp.full((nr, 1, 1), 2**30, jnp.int32),
    )

    tp2 = t_p.reshape(nr, 1)
    lse2 = lse.reshape(nr, 1)
    ic2 = icut.reshape(nr, 1)
    vi = lax.broadcasted_iota(jnp.int32, (nr, v), 1)
    keep = jnp.logical_or(x > tp2, jnp.logical_and(x == tp2, vi <= ic2))
    o_ref[...] = jnp.where(keep, x - lse2, NEG)


@jax.jit
def kernel(logits):
    b, h, v = logits.shape
    n = b * h
    nrows = (v + 127) // 128
    nrows = ((nrows + 8 * NSUB - 1) // (8 * NSUB)) * (8 * NSUB)
    x2 = logits.reshape(n, v)
    rb = RB if n % RB == 0 else 1
    out = pl.pallas_call(
        functools.partial(_fused_kernel, nr=rb, v=v, nrows=nrows, ncols=128),
        grid=(n // rb,),
        in_specs=[pl.BlockSpec((rb, v), lambda i: (i, 0))],
        out_specs=pl.BlockSpec((rb, v), lambda i: (i, 0)),
        out_shape=jax.ShapeDtypeStruct((n, v), jnp.float32),
        scratch_shapes=[
            pltpu.VMEM((rb * NSUB, nrows // NSUB, 128), jnp.float32),
            pltpu.VMEM((rb * NSUB, RCAP, 128), jnp.float32),
            pltpu.VMEM((rb * NSUB, RCAP, 128), jnp.float32),
        ],
    )(x2)
    return out.reshape(b, h, v)
